# baseline (speedup 1.0000x reference)
import functools

import jax
import jax.numpy as jnp
import numpy as np
from jax import lax
from jax.experimental import pallas as pl
from jax.experimental.pallas import tpu as pltpu
from jax.experimental.pallas import tpu_sc as plsc

DIM = 1024
HEADS = 4
NUM_KEYS = 256
TOPK = 32
SUB_D = 128
BN_EPS = 1e-5
N_CAND = 128

_TOKENS = 4096
_TILE = 256
_N_TILES = _TOKENS // _TILE


def _candidate_onehots():
    order = sorted(((i + 1) * (j + 1), i, j) for i in range(TOPK) for j in range(TOPK))
    pi = np.array([i for _, i, _ in order[:N_CAND]], dtype=np.int64)
    pj = np.array([j for _, _, j in order[:N_CAND]], dtype=np.int64)
    ohi = (np.arange(TOPK)[:, None] == pi[None, :]).astype(np.float32)
    ohj = (np.arange(TOPK)[:, None] == pj[None, :]).astype(np.float32)
    return ohi, ohj


_OHI, _OHJ = _candidate_onehots()


def _k1_body(x_ref, wq_ref, mask_ref, q_ref, stats_ref):
    i = pl.program_id(0)
    xt = x_ref[...].astype(jnp.bfloat16)
    q = lax.dot_general(xt, wq_ref[...].astype(jnp.bfloat16),
                        (((1,), (1,)), ((), ())),
                        preferred_element_type=jnp.float32)
    q_ref[...] = q
    m = mask_ref[:, 0:1]
    qm = q * m

    @pl.when(i == 0)
    def _():
        stats_ref[...] = jnp.zeros((8, DIM), jnp.float32)

    stats_ref[0:1, :] = stats_ref[0:1, :] + jnp.sum(qm, axis=0, keepdims=True)
    stats_ref[1:2, :] = stats_ref[1:2, :] + jnp.sum(qm * q, axis=0, keepdims=True)


def _run_k1(xf, w_q, maskf):
    return pl.pallas_call(
        _k1_body,
        grid=(_N_TILES,),
        in_specs=[
            pl.BlockSpec((_TILE, DIM), lambda i: (i, 0)),
            pl.BlockSpec((DIM, DIM), lambda i: (0, 0)),
            pl.BlockSpec((_TILE, 128), lambda i: (i, 0)),
        ],
        out_specs=[
            pl.BlockSpec((_TILE, DIM), lambda i: (i, 0)),
            pl.BlockSpec((8, DIM), lambda i: (0, 0)),
        ],
        out_shape=[
            jax.ShapeDtypeStruct((_TOKENS, DIM), jnp.float32),
            jax.ShapeDtypeStruct((8, DIM), jnp.float32),
        ],
    )(xf, w_q, maskf)


def _top32(work_ref, n_lanes):
    rows = work_ref.shape[0]
    lane = lax.broadcasted_iota(jnp.int32, (rows, n_lanes), 1)
    kcol = lax.broadcasted_iota(jnp.int32, (rows, TOPK), 1)

    def body(k, car):
        s_acc, i_acc = car
        w = work_ref[...]
        mx = jnp.max(w, axis=1, keepdims=True)
        am = jnp.min(jnp.where(w == mx, lane, n_lanes), axis=1, keepdims=True)
        work_ref[...] = jnp.where(lane == am, -3.0e38, w)
        s_acc = jnp.where(kcol == k, mx, s_acc)
        i_acc = jnp.where(kcol == k, am.astype(jnp.float32), i_acc)
        return s_acc, i_acc

    z = jnp.zeros((rows, TOPK), jnp.float32)
    return lax.fori_loop(0, TOPK, body, (z, z))


def _top32_with_payload(work_ref, payload):
    rows = work_ref.shape[0]
    n_lanes = N_CAND
    lane = lax.broadcasted_iota(jnp.int32, (rows, n_lanes), 1)
    kcol = lax.broadcasted_iota(jnp.int32, (rows, TOPK), 1)

    def body(k, car):
        s_acc, v_acc = car
        w = work_ref[...]
        mx = jnp.max(w, axis=1, keepdims=True)
        am = jnp.min(jnp.where(w == mx, lane, n_lanes), axis=1, keepdims=True)
        sel = lane == am
        work_ref[...] = jnp.where(sel, -3.0e38, w)
        vk = jnp.sum(jnp.where(sel, payload, 0.0), axis=1, keepdims=True)
        s_acc = jnp.where(kcol == k, mx, s_acc)
        v_acc = jnp.where(kcol == k, vk, v_acc)
        return s_acc, v_acc

    z = jnp.zeros((rows, TOPK), jnp.float32)
    return lax.fori_loop(0, TOPK, body, (z, z))


def _mm(a, b):
    return lax.dot_general(a, b, (((1,), (0,)), ((), ())),
                           preferred_element_type=jnp.float32,
                           precision=lax.Precision.HIGHEST)


def _k2_body(q_ref, stats_ref, gam_ref, bet_ref, mask_ref, keys_ref,
             ohi_ref, ohj_ref, cnt_ref, attn_ref, vi_ref, work_ref):
    cnt = cnt_ref[0, 0]
    s0 = stats_ref[0:1, :]
    s1 = stats_ref[1:2, :]
    mean = s0 / cnt
    var = s1 / cnt - mean * mean
    q = q_ref[...]
    m = mask_ref[:, 0:1]
    normed = (q - mean) / jnp.sqrt(var + BN_EPS) * gam_ref[...] + bet_ref[...]
    qn = jnp.where(m > 0, normed, q)
    ohi = ohi_ref[...]
    ohj = ohj_ref[...]

    for h in range(HEADS):
        tops = []
        for p in range(2):
            qhp = qn[:, p * 512 + h * 128:p * 512 + (h + 1) * 128]
            khp = keys_ref[h, p]
            work_ref[...] = lax.dot_general(
                qhp.astype(jnp.bfloat16), khp.astype(jnp.bfloat16),
                (((1,), (1,)), ((), ())),
                preferred_element_type=jnp.float32)
            tops.append(_top32(work_ref, NUM_KEYS))
        (sx, ix), (sy, iy) = tops
        csum = _mm(sx, ohi) + _mm(sy, ohj)
        cidx = _mm(ix * float(NUM_KEYS), ohi) + _mm(iy, ohj)
        work_ref[:, 0:N_CAND] = csum
        vs, vv = _top32_with_payload(work_ref.at[:, 0:N_CAND], cidx)
        mx0 = jnp.max(vs, axis=1, keepdims=True)
        e = jnp.exp(vs - mx0)
        a = e / jnp.sum(e, axis=1, keepdims=True)
        attn_ref[:, h * TOPK:(h + 1) * TOPK] = a
        vi_ref[:, h * TOPK:(h + 1) * TOPK] = vv.astype(jnp.int32)


def _run_k2(q, stats, gamma, beta, maskf, keys_t, ohi, ohj, cnt):
    return pl.pallas_call(
        _k2_body,
        grid=(_N_TILES,),
        in_specs=[
            pl.BlockSpec((_TILE, DIM), lambda i: (i, 0)),
            pl.BlockSpec((8, DIM), lambda i: (0, 0)),
            pl.BlockSpec((1, DIM), lambda i: (0, 0)),
            pl.BlockSpec((1, DIM), lambda i: (0, 0)),
            pl.BlockSpec((_TILE, 128), lambda i: (i, 0)),
            pl.BlockSpec((HEADS, 2, NUM_KEYS, SUB_D), lambda i: (0, 0, 0, 0)),
            pl.BlockSpec((TOPK, N_CAND), lambda i: (0, 0)),
            pl.BlockSpec((TOPK, N_CAND), lambda i: (0, 0)),
            pl.BlockSpec(memory_space=pltpu.SMEM),
        ],
        out_specs=[
            pl.BlockSpec((_TILE, HEADS * TOPK), lambda i: (i, 0)),
            pl.BlockSpec((_TILE, HEADS * TOPK), lambda i: (i, 0)),
        ],
        out_shape=[
            jax.ShapeDtypeStruct((_TOKENS, HEADS * TOPK), jnp.float32),
            jax.ShapeDtypeStruct((_TOKENS, HEADS * TOPK), jnp.int32),
        ],
        scratch_shapes=[pltpu.VMEM((_TILE, NUM_KEYS), jnp.float32)],
    )(q, stats, gamma, beta, maskf, keys_t, ohi, ohj, cnt)


_NC = 2
_NS = 16
_NW = _NC * _NS
_TPW = _TOKENS // _NW
_CH = 8
_NCH = (HEADS * TOPK) // _CH
_GT = _TPW * _NCH
_LSEG = DIM // 16


_TBLK = 32
_NBLK = _TPW // _TBLK
_NBUF = 4
_GBLK = _TBLK * _NCH


def _k3_body(vals_hbm, vi_hbm, at_hbm, out_hbm,
             idx_v, w_v, rows_v, acc_v, sem0, sem1, sem2, sem3):
    wid = lax.axis_index("s") * _NC + lax.axis_index("c")
    base = wid * _TPW
    sems = (sem0, sem1, sem2, sem3)

    def start_gather(tok, kc, b):
        pltpu.async_copy(vals_hbm.at[idx_v.at[tok, pl.ds(kc * _CH, _CH)]],
                         rows_v.at[b], sems[b])

    for blk in range(_NBLK):
        tok0 = base + blk * _TBLK
        pltpu.sync_copy(vi_hbm.at[pl.ds(tok0, _TBLK)], idx_v)
        pltpu.sync_copy(at_hbm.at[pl.ds(tok0, _TBLK)], w_v)
        for b in range(_NBUF):
            start_gather(b // _NCH, b % _NCH, b)

        @pl.loop(0, _GBLK, step=_NBUF)
        def _(g0):
            for b in range(_NBUF):
                g = g0 + b
                tok = g // _NCH
                kc = g % _NCH
                tm = lax.rem(tok, 8)
                pltpu.make_async_copy(
                    vals_hbm.at[idx_v.at[tok, pl.ds(kc * _CH, _CH)]],
                    rows_v.at[b], sems[b]).wait()

                @pl.when(kc == 0)
                def _():
                    @pl.loop(0, _LSEG, unroll=8)
                    def _(l):
                        acc_v[tm, pl.ds(l * 16, 16)] = jnp.zeros((16,),
                                                                 jnp.float32)

                wv = w_v[tok, pl.ds((kc // 2) * 16, 16)]
                for rr in range(_CH):
                    w = wv[(b % 2) * _CH + rr]

                    @pl.loop(0, _LSEG, unroll=8)
                    def _(l):
                        seg = pl.ds(l * 16, 16)
                        plsc.addupdate(acc_v.at[tm, seg],
                                       w * rows_v[b, rr, seg])

                @pl.when((kc == _NCH - 1) & (tm == 7))
                def _():
                    off = pl.multiple_of(tok0 + tok - 7, 8)
                    pltpu.sync_copy(acc_v, out_hbm.at[pl.ds(off, 8)])

                @pl.when(g + _NBUF < _GBLK)
                def _():
                    g2 = g + _NBUF
                    start_gather(g2 // _NCH, lax.rem(g2, _NCH), b)


def _run_k3(values, vi3, attn):
    mesh = plsc.VectorSubcoreMesh(core_axis_name="c", subcore_axis_name="s")
    k = functools.partial(
        pl.kernel,
        out_type=jax.ShapeDtypeStruct((_TOKENS, DIM), jnp.float32),
        mesh=mesh,
        scratch_types=[
            pltpu.VMEM((_TBLK, HEADS * TOPK), jnp.int32),
            pltpu.VMEM((_TBLK, HEADS * TOPK), jnp.float32),
            pltpu.VMEM((_NBUF, _CH, DIM), jnp.float32),
            pltpu.VMEM((8, DIM), jnp.float32),
            pltpu.SemaphoreType.DMA,
            pltpu.SemaphoreType.DMA,
            pltpu.SemaphoreType.DMA,
            pltpu.SemaphoreType.DMA,
        ],
    )(_k3_body)
    return k(values, vi3, attn)


def kernel(x, input_mask, w_q, bn_gamma, bn_beta, keys, values):
    b, t = x.shape[0], x.shape[1]
    xf = x.reshape(b * t, DIM)
    maskcol = input_mask.reshape(b * t, 1).astype(jnp.float32)
    maskf = jnp.broadcast_to(maskcol, (b * t, 128))
    cnt = jnp.maximum(jnp.sum(maskcol), 1.0).reshape(1, 1)
    keys_t = jnp.transpose(keys, (0, 2, 1, 3))
    ohi = jnp.asarray(_OHI)
    ohj = jnp.asarray(_OHJ)

    q, stats = _run_k1(xf, w_q, maskf)
    attn, vi = _run_k2(q, stats, bn_gamma.reshape(1, DIM),
                       bn_beta.reshape(1, DIM), maskf, keys_t, ohi, ohj, cnt)
    out = _run_k3(values, vi, attn)
    return out.reshape(b, t, DIM)


# speedup vs baseline: 1.8501x; 1.8501x over previous
import functools

import jax
import jax.numpy as jnp
import numpy as np
from jax import lax
from jax.experimental import pallas as pl
from jax.experimental.pallas import tpu as pltpu
from jax.experimental.pallas import tpu_sc as plsc

DIM = 1024
HEADS = 4
NUM_KEYS = 256
TOPK = 32
SUB_D = 128
BN_EPS = 1e-5
N_CAND = 128

_TOKENS = 4096
_TILE = 256
_N_TILES = _TOKENS // _TILE


def _candidate_onehots():
    order = sorted(((i + 1) * (j + 1), i, j) for i in range(TOPK) for j in range(TOPK))
    pi = np.array([i for _, i, _ in order[:N_CAND]], dtype=np.int64)
    pj = np.array([j for _, _, j in order[:N_CAND]], dtype=np.int64)
    ohi = (np.arange(TOPK)[:, None] == pi[None, :]).astype(np.float32)
    ohj = (np.arange(TOPK)[:, None] == pj[None, :]).astype(np.float32)
    return ohi, ohj


_OHI, _OHJ = _candidate_onehots()


def _k1_body(x_ref, wq_ref, mask_ref, q_ref, stats_ref):
    i = pl.program_id(0)
    xt = x_ref[...].astype(jnp.bfloat16)
    q = lax.dot_general(xt, wq_ref[...].astype(jnp.bfloat16),
                        (((1,), (1,)), ((), ())),
                        preferred_element_type=jnp.float32)
    q_ref[...] = q
    m = mask_ref[:, 0:1]
    qm = q * m

    @pl.when(i == 0)
    def _():
        stats_ref[...] = jnp.zeros((8, DIM), jnp.float32)

    stats_ref[0:1, :] = stats_ref[0:1, :] + jnp.sum(qm, axis=0, keepdims=True)
    stats_ref[1:2, :] = stats_ref[1:2, :] + jnp.sum(qm * q, axis=0, keepdims=True)


def _run_k1(xf, w_q, maskf):
    return pl.pallas_call(
        _k1_body,
        grid=(_N_TILES,),
        in_specs=[
            pl.BlockSpec((_TILE, DIM), lambda i: (i, 0)),
            pl.BlockSpec((DIM, DIM), lambda i: (0, 0)),
            pl.BlockSpec((_TILE, 128), lambda i: (i, 0)),
        ],
        out_specs=[
            pl.BlockSpec((_TILE, DIM), lambda i: (i, 0)),
            pl.BlockSpec((8, DIM), lambda i: (0, 0)),
        ],
        out_shape=[
            jax.ShapeDtypeStruct((_TOKENS, DIM), jnp.float32),
            jax.ShapeDtypeStruct((8, DIM), jnp.float32),
        ],
    )(xf, w_q, maskf)


def _top32(work_ref, n_lanes):
    rows = work_ref.shape[0]
    lane = lax.broadcasted_iota(jnp.int32, (rows, n_lanes), 1)
    kcol = lax.broadcasted_iota(jnp.int32, (rows, TOPK), 1)

    def body(k, car):
        s_acc, i_acc = car
        w = work_ref[...]
        mx = jnp.max(w, axis=1, keepdims=True)
        am = jnp.min(jnp.where(w == mx, lane, n_lanes), axis=1, keepdims=True)
        work_ref[...] = jnp.where(lane == am, -3.0e38, w)
        s_acc = jnp.where(kcol == k, mx, s_acc)
        i_acc = jnp.where(kcol == k, am.astype(jnp.float32), i_acc)
        return s_acc, i_acc

    z = jnp.zeros((rows, TOPK), jnp.float32)
    return lax.fori_loop(0, TOPK, body, (z, z))


def _top32_with_payload(work_ref, payload):
    rows = work_ref.shape[0]
    n_lanes = N_CAND
    lane = lax.broadcasted_iota(jnp.int32, (rows, n_lanes), 1)
    kcol = lax.broadcasted_iota(jnp.int32, (rows, TOPK), 1)

    def body(k, car):
        s_acc, v_acc = car
        w = work_ref[...]
        mx = jnp.max(w, axis=1, keepdims=True)
        am = jnp.min(jnp.where(w == mx, lane, n_lanes), axis=1, keepdims=True)
        sel = lane == am
        work_ref[...] = jnp.where(sel, -3.0e38, w)
        vk = jnp.sum(jnp.where(sel, payload, 0.0), axis=1, keepdims=True)
        s_acc = jnp.where(kcol == k, mx, s_acc)
        v_acc = jnp.where(kcol == k, vk, v_acc)
        return s_acc, v_acc

    z = jnp.zeros((rows, TOPK), jnp.float32)
    return lax.fori_loop(0, TOPK, body, (z, z))


def _mm(a, b):
    return lax.dot_general(a, b, (((1,), (0,)), ((), ())),
                           preferred_element_type=jnp.float32,
                           precision=lax.Precision.HIGHEST)


def _k2_body(q_ref, stats_ref, gam_ref, bet_ref, mask_ref, keys_ref,
             ohi_ref, ohj_ref, cnt_ref, attn_ref, vi_ref, work_ref):
    cnt = cnt_ref[0, 0]
    s0 = stats_ref[0:1, :]
    s1 = stats_ref[1:2, :]
    mean = s0 / cnt
    var = s1 / cnt - mean * mean
    q = q_ref[...]
    m = mask_ref[:, 0:1]
    normed = (q - mean) / jnp.sqrt(var + BN_EPS) * gam_ref[...] + bet_ref[...]
    qn = jnp.where(m > 0, normed, q)
    ohi = ohi_ref[...]
    ohj = ohj_ref[...]

    for h in range(HEADS):
        tops = []
        for p in range(2):
            qhp = qn[:, p * 512 + h * 128:p * 512 + (h + 1) * 128]
            khp = keys_ref[h, p]
            work_ref[...] = lax.dot_general(
                qhp.astype(jnp.bfloat16), khp.astype(jnp.bfloat16),
                (((1,), (1,)), ((), ())),
                preferred_element_type=jnp.float32)
            tops.append(_top32(work_ref, NUM_KEYS))
        (sx, ix), (sy, iy) = tops
        csum = _mm(sx, ohi) + _mm(sy, ohj)
        cidx = _mm(ix * float(NUM_KEYS), ohi) + _mm(iy, ohj)
        work_ref[:, 0:N_CAND] = csum
        vs, vv = _top32_with_payload(work_ref.at[:, 0:N_CAND], cidx)
        mx0 = jnp.max(vs, axis=1, keepdims=True)
        e = jnp.exp(vs - mx0)
        a = e / jnp.sum(e, axis=1, keepdims=True)
        attn_ref[:, h * TOPK:(h + 1) * TOPK] = a
        vi_ref[:, h * TOPK:(h + 1) * TOPK] = vv.astype(jnp.int32)


def _run_k2(q, stats, gamma, beta, maskf, keys_t, ohi, ohj, cnt):
    return pl.pallas_call(
        _k2_body,
        grid=(_N_TILES,),
        in_specs=[
            pl.BlockSpec((_TILE, DIM), lambda i: (i, 0)),
            pl.BlockSpec((8, DIM), lambda i: (0, 0)),
            pl.BlockSpec((1, DIM), lambda i: (0, 0)),
            pl.BlockSpec((1, DIM), lambda i: (0, 0)),
            pl.BlockSpec((_TILE, 128), lambda i: (i, 0)),
            pl.BlockSpec((HEADS, 2, NUM_KEYS, SUB_D), lambda i: (0, 0, 0, 0)),
            pl.BlockSpec((TOPK, N_CAND), lambda i: (0, 0)),
            pl.BlockSpec((TOPK, N_CAND), lambda i: (0, 0)),
            pl.BlockSpec(memory_space=pltpu.SMEM),
        ],
        out_specs=[
            pl.BlockSpec((_TILE, HEADS * TOPK), lambda i: (i, 0)),
            pl.BlockSpec((_TILE, HEADS * TOPK), lambda i: (i, 0)),
        ],
        out_shape=[
            jax.ShapeDtypeStruct((_TOKENS, HEADS * TOPK), jnp.float32),
            jax.ShapeDtypeStruct((_TOKENS, HEADS * TOPK), jnp.int32),
        ],
        scratch_shapes=[pltpu.VMEM((_TILE, NUM_KEYS), jnp.float32)],
    )(q, stats, gamma, beta, maskf, keys_t, ohi, ohj, cnt)


_NC = 2
_NS = 16
_NW = _NC * _NS
_TPW = _TOKENS // _NW
_CH = 8
_NCH = (HEADS * TOPK) // _CH
_GT = _TPW * _NCH
_LSEG = DIM // 16


_TBLK = 32
_NBLK = _TPW // _TBLK
_NBUF = 4
_GBLK = _TBLK * _NCH


def _k3_body(vals_hbm, vi_hbm, at_hbm, out_hbm,
             idx_v, w_v, rows_v, acc_v, sem0, sem1, sem2, sem3):
    wid = lax.axis_index("s") * _NC + lax.axis_index("c")
    base = wid * _TPW
    sems = (sem0, sem1, sem2, sem3)

    def start_gather(tok, kc, b):
        pltpu.async_copy(vals_hbm.at[idx_v.at[tok, pl.ds(kc * _CH, _CH)]],
                         rows_v.at[b], sems[b])

    for blk in range(_NBLK):
        tok0 = base + blk * _TBLK
        pltpu.sync_copy(vi_hbm.at[pl.ds(tok0, _TBLK)], idx_v)
        pltpu.sync_copy(at_hbm.at[pl.ds(tok0, _TBLK)], w_v)
        for b in range(_NBUF):
            start_gather(b // _NCH, b % _NCH, b)

        @pl.loop(0, _GBLK, step=_NBUF)
        def _(g0):
            for b in range(_NBUF):
                g = g0 + b
                tok = g // _NCH
                kc = g % _NCH
                tm = lax.rem(tok, 8)
                pltpu.make_async_copy(
                    vals_hbm.at[idx_v.at[tok, pl.ds(kc * _CH, _CH)]],
                    rows_v.at[b], sems[b]).wait()

                @pl.when(kc == 0)
                def _():
                    @pl.loop(0, _LSEG, unroll=8)
                    def _(l):
                        acc_v[tm, pl.ds(l * 16, 16)] = jnp.zeros((16,),
                                                                 jnp.float32)

                wv = w_v[tok, pl.ds((kc // 2) * 16, 16)]
                wlist = [wv[(b % 2) * _CH + rr] for rr in range(_CH)]

                @pl.loop(0, _LSEG, unroll=4)
                def _(l):
                    seg = pl.ds(l * 16, 16)
                    v = wlist[0] * rows_v[b, 0, seg]
                    for rr in range(1, _CH):
                        v = v + wlist[rr] * rows_v[b, rr, seg]
                    plsc.addupdate(acc_v.at[tm, seg], v)

                @pl.when((kc == _NCH - 1) & (tm == 7))
                def _():
                    off = pl.multiple_of(tok0 + tok - 7, 8)
                    pltpu.sync_copy(acc_v, out_hbm.at[pl.ds(off, 8)])

                @pl.when(g + _NBUF < _GBLK)
                def _():
                    g2 = g + _NBUF
                    start_gather(g2 // _NCH, lax.rem(g2, _NCH), b)


def _run_k3(values, vi3, attn):
    mesh = plsc.VectorSubcoreMesh(core_axis_name="c", subcore_axis_name="s")
    k = functools.partial(
        pl.kernel,
        out_type=jax.ShapeDtypeStruct((_TOKENS, DIM), jnp.float32),
        mesh=mesh,
        scratch_types=[
            pltpu.VMEM((_TBLK, HEADS * TOPK), jnp.int32),
            pltpu.VMEM((_TBLK, HEADS * TOPK), jnp.float32),
            pltpu.VMEM((_NBUF, _CH, DIM), jnp.float32),
            pltpu.VMEM((8, DIM), jnp.float32),
            pltpu.SemaphoreType.DMA,
            pltpu.SemaphoreType.DMA,
            pltpu.SemaphoreType.DMA,
            pltpu.SemaphoreType.DMA,
        ],
    )(_k3_body)
    return k(values, vi3, attn)


def kernel(x, input_mask, w_q, bn_gamma, bn_beta, keys, values):
    b, t = x.shape[0], x.shape[1]
    xf = x.reshape(b * t, DIM)
    maskcol = input_mask.reshape(b * t, 1).astype(jnp.float32)
    maskf = jnp.broadcast_to(maskcol, (b * t, 128))
    cnt = jnp.maximum(jnp.sum(maskcol), 1.0).reshape(1, 1)
    keys_t = jnp.transpose(keys, (0, 2, 1, 3))
    ohi = jnp.asarray(_OHI)
    ohj = jnp.asarray(_OHJ)

    q, stats = _run_k1(xf, w_q, maskf)
    attn, vi = _run_k2(q, stats, bn_gamma.reshape(1, DIM),
                       bn_beta.reshape(1, DIM), maskf, keys_t, ohi, ohj, cnt)
    out = _run_k3(values, vi, attn)
    return out.reshape(b, t, DIM)


# speedup vs baseline: 2.3937x; 1.2938x over previous
import functools

import jax
import jax.numpy as jnp
import numpy as np
from jax import lax
from jax.experimental import pallas as pl
from jax.experimental.pallas import tpu as pltpu
from jax.experimental.pallas import tpu_sc as plsc

DIM = 1024
HEADS = 4
NUM_KEYS = 256
TOPK = 32
SUB_D = 128
BN_EPS = 1e-5
N_CAND = 128

_TOKENS = 4096
_TILE = 256
_N_TILES = _TOKENS // _TILE


def _candidate_onehots():
    order = sorted(((i + 1) * (j + 1), i, j) for i in range(TOPK) for j in range(TOPK))
    pi = np.array([i for _, i, _ in order[:N_CAND]], dtype=np.int64)
    pj = np.array([j for _, _, j in order[:N_CAND]], dtype=np.int64)
    ohi = (np.arange(TOPK)[:, None] == pi[None, :]).astype(np.float32)
    ohj = (np.arange(TOPK)[:, None] == pj[None, :]).astype(np.float32)
    return ohi, ohj


_OHI, _OHJ = _candidate_onehots()


def _k1_body(x_ref, wq_ref, mask_ref, q_ref, stats_ref):
    i = pl.program_id(0)
    xt = x_ref[...].astype(jnp.bfloat16)
    q = lax.dot_general(xt, wq_ref[...].astype(jnp.bfloat16),
                        (((1,), (1,)), ((), ())),
                        preferred_element_type=jnp.float32)
    q_ref[...] = q
    m = mask_ref[:, 0:1]
    qm = q * m

    @pl.when(i == 0)
    def _():
        stats_ref[...] = jnp.zeros((8, DIM), jnp.float32)

    stats_ref[0:1, :] = stats_ref[0:1, :] + jnp.sum(qm, axis=0, keepdims=True)
    stats_ref[1:2, :] = stats_ref[1:2, :] + jnp.sum(qm * q, axis=0, keepdims=True)


def _run_k1(xf, w_q, maskf):
    return pl.pallas_call(
        _k1_body,
        grid=(_N_TILES,),
        in_specs=[
            pl.BlockSpec((_TILE, DIM), lambda i: (i, 0)),
            pl.BlockSpec((DIM, DIM), lambda i: (0, 0)),
            pl.BlockSpec((_TILE, 128), lambda i: (i, 0)),
        ],
        out_specs=[
            pl.BlockSpec((_TILE, DIM), lambda i: (i, 0)),
            pl.BlockSpec((8, DIM), lambda i: (0, 0)),
        ],
        out_shape=[
            jax.ShapeDtypeStruct((_TOKENS, DIM), jnp.float32),
            jax.ShapeDtypeStruct((8, DIM), jnp.float32),
        ],
    )(xf, w_q, maskf)


def _top32(work_ref, n_lanes):
    rows = work_ref.shape[0]
    lane = lax.broadcasted_iota(jnp.int32, (rows, n_lanes), 1)
    kcol = lax.broadcasted_iota(jnp.int32, (rows, TOPK), 1)

    def body(k, car):
        s_acc, i_acc = car
        w = work_ref[...]
        mx = jnp.max(w, axis=1, keepdims=True)
        am = jnp.min(jnp.where(w == mx, lane, n_lanes), axis=1, keepdims=True)
        work_ref[...] = jnp.where(lane == am, -3.0e38, w)
        s_acc = jnp.where(kcol == k, mx, s_acc)
        i_acc = jnp.where(kcol == k, am.astype(jnp.float32), i_acc)
        return s_acc, i_acc

    z = jnp.zeros((rows, TOPK), jnp.float32)
    return lax.fori_loop(0, TOPK, body, (z, z))


def _top32_with_payload(work_ref, payload):
    rows = work_ref.shape[0]
    n_lanes = N_CAND
    lane = lax.broadcasted_iota(jnp.int32, (rows, n_lanes), 1)
    kcol = lax.broadcasted_iota(jnp.int32, (rows, TOPK), 1)

    def body(k, car):
        s_acc, v_acc = car
        w = work_ref[...]
        mx = jnp.max(w, axis=1, keepdims=True)
        am = jnp.min(jnp.where(w == mx, lane, n_lanes), axis=1, keepdims=True)
        sel = lane == am
        work_ref[...] = jnp.where(sel, -3.0e38, w)
        vk = jnp.sum(jnp.where(sel, payload, 0.0), axis=1, keepdims=True)
        s_acc = jnp.where(kcol == k, mx, s_acc)
        v_acc = jnp.where(kcol == k, vk, v_acc)
        return s_acc, v_acc

    z = jnp.zeros((rows, TOPK), jnp.float32)
    return lax.fori_loop(0, TOPK, body, (z, z))


def _mm(a, b):
    return lax.dot_general(a, b, (((1,), (0,)), ((), ())),
                           preferred_element_type=jnp.float32,
                           precision=lax.Precision.HIGHEST)


def _k2_body(q_ref, stats_ref, gam_ref, bet_ref, mask_ref, keys_ref,
             ohi_ref, ohj_ref, cnt_ref, attn_ref, vi_ref, work_ref):
    cnt = cnt_ref[0, 0]
    s0 = stats_ref[0:1, :]
    s1 = stats_ref[1:2, :]
    mean = s0 / cnt
    var = s1 / cnt - mean * mean
    q = q_ref[...]
    m = mask_ref[:, 0:1]
    normed = (q - mean) / jnp.sqrt(var + BN_EPS) * gam_ref[...] + bet_ref[...]
    qn = jnp.where(m > 0, normed, q)
    ohi = ohi_ref[...]
    ohj = ohj_ref[...]

    for h in range(HEADS):
        tops = []
        for p in range(2):
            qhp = qn[:, p * 512 + h * 128:p * 512 + (h + 1) * 128]
            khp = keys_ref[h, p]
            work_ref[...] = lax.dot_general(
                qhp.astype(jnp.bfloat16), khp.astype(jnp.bfloat16),
                (((1,), (1,)), ((), ())),
                preferred_element_type=jnp.float32)
            tops.append(_top32(work_ref, NUM_KEYS))
        (sx, ix), (sy, iy) = tops
        csum = _mm(sx, ohi) + _mm(sy, ohj)
        cidx = _mm(ix * float(NUM_KEYS), ohi) + _mm(iy, ohj)
        work_ref[:, 0:N_CAND] = csum
        vs, vv = _top32_with_payload(work_ref.at[:, 0:N_CAND], cidx)
        mx0 = jnp.max(vs, axis=1, keepdims=True)
        e = jnp.exp(vs - mx0)
        a = e / jnp.sum(e, axis=1, keepdims=True)
        attn_ref[:, h * TOPK:(h + 1) * TOPK] = a
        vi_ref[:, h * TOPK:(h + 1) * TOPK] = vv.astype(jnp.int32)


def _run_k2(q, stats, gamma, beta, maskf, keys_t, ohi, ohj, cnt):
    n = q.shape[0]
    nt = n // _TILE
    return pl.pallas_call(
        _k2_body,
        grid=(nt,),
        in_specs=[
            pl.BlockSpec((_TILE, DIM), lambda i: (i, 0)),
            pl.BlockSpec((8, DIM), lambda i: (0, 0)),
            pl.BlockSpec((1, DIM), lambda i: (0, 0)),
            pl.BlockSpec((1, DIM), lambda i: (0, 0)),
            pl.BlockSpec((_TILE, 128), lambda i: (i, 0)),
            pl.BlockSpec((HEADS, 2, NUM_KEYS, SUB_D), lambda i: (0, 0, 0, 0)),
            pl.BlockSpec((TOPK, N_CAND), lambda i: (0, 0)),
            pl.BlockSpec((TOPK, N_CAND), lambda i: (0, 0)),
            pl.BlockSpec(memory_space=pltpu.SMEM),
        ],
        out_specs=[
            pl.BlockSpec((_TILE, HEADS * TOPK), lambda i: (i, 0)),
            pl.BlockSpec((_TILE, HEADS * TOPK), lambda i: (i, 0)),
        ],
        out_shape=[
            jax.ShapeDtypeStruct((n, HEADS * TOPK), jnp.float32),
            jax.ShapeDtypeStruct((n, HEADS * TOPK), jnp.int32),
        ],
        scratch_shapes=[pltpu.VMEM((_TILE, NUM_KEYS), jnp.float32)],
    )(q, stats, gamma, beta, maskf, keys_t, ohi, ohj, cnt)


_NC = 2
_NS = 16
_NW = _NC * _NS
_TPW = _TOKENS // _NW
_CH = 8
_NCH = (HEADS * TOPK) // _CH
_GT = _TPW * _NCH
_LSEG = DIM // 16


_TBLK = 32
_NBLK = _TPW // _TBLK
_NBUF = 4
_GBLK = _TBLK * _NCH


def _make_k3_body(tpw, nblk):
  def _k3_body(vals_hbm, vi_hbm, at_hbm, out_hbm,
               idx_v, w_v, rows_v, acc_v, sem0, sem1, sem2, sem3):
    wid = lax.axis_index("s") * _NC + lax.axis_index("c")
    base = wid * tpw
    sems = (sem0, sem1, sem2, sem3)

    def start_gather(tok, kc, b):
        pltpu.async_copy(vals_hbm.at[idx_v.at[tok, pl.ds(kc * _CH, _CH)]],
                         rows_v.at[b], sems[b])

    for blk in range(nblk):
        tok0 = base + blk * _TBLK
        pltpu.sync_copy(vi_hbm.at[pl.ds(tok0, _TBLK)], idx_v)
        pltpu.sync_copy(at_hbm.at[pl.ds(tok0, _TBLK)], w_v)
        for b in range(_NBUF):
            start_gather(b // _NCH, b % _NCH, b)

        @pl.loop(0, _GBLK, step=_NBUF)
        def _(g0):
            for b in range(_NBUF):
                g = g0 + b
                tok = g // _NCH
                kc = g % _NCH
                tm = lax.rem(tok, 8)
                pltpu.make_async_copy(
                    vals_hbm.at[idx_v.at[tok, pl.ds(kc * _CH, _CH)]],
                    rows_v.at[b], sems[b]).wait()

                @pl.when(kc == 0)
                def _():
                    @pl.loop(0, _LSEG, unroll=8)
                    def _(l):
                        acc_v[tm, pl.ds(l * 16, 16)] = jnp.zeros((16,),
                                                                 jnp.float32)

                wv = w_v[tok, pl.ds((kc // 2) * 16, 16)]
                wlist = [wv[(b % 2) * _CH + rr] for rr in range(_CH)]

                @pl.loop(0, _LSEG, unroll=4)
                def _(l):
                    seg = pl.ds(l * 16, 16)
                    v = wlist[0] * rows_v[b, 0, seg]
                    for rr in range(1, _CH):
                        v = v + wlist[rr] * rows_v[b, rr, seg]
                    plsc.addupdate(acc_v.at[tm, seg], v)

                @pl.when((kc == _NCH - 1) & (tm == 7))
                def _():
                    off = pl.multiple_of(tok0 + tok - 7, 8)
                    pltpu.sync_copy(acc_v, out_hbm.at[pl.ds(off, 8)])

                @pl.when(g + _NBUF < _GBLK)
                def _():
                    g2 = g + _NBUF
                    start_gather(g2 // _NCH, lax.rem(g2, _NCH), b)

  return _k3_body


def _run_k3(values, vi3, attn):
    n = vi3.shape[0]
    tpw = n // _NW
    nblk = tpw // _TBLK
    mesh = plsc.VectorSubcoreMesh(core_axis_name="c", subcore_axis_name="s")
    k = functools.partial(
        pl.kernel,
        out_type=jax.ShapeDtypeStruct((n, DIM), jnp.float32),
        mesh=mesh,
        scratch_types=[
            pltpu.VMEM((_TBLK, HEADS * TOPK), jnp.int32),
            pltpu.VMEM((_TBLK, HEADS * TOPK), jnp.float32),
            pltpu.VMEM((_NBUF, _CH, DIM), jnp.float32),
            pltpu.VMEM((8, DIM), jnp.float32),
            pltpu.SemaphoreType.DMA,
            pltpu.SemaphoreType.DMA,
            pltpu.SemaphoreType.DMA,
            pltpu.SemaphoreType.DMA,
        ],
    )(_make_k3_body(tpw, nblk))
    return k(values, vi3, attn)


def kernel(x, input_mask, w_q, bn_gamma, bn_beta, keys, values):
    b, t = x.shape[0], x.shape[1]
    xf = x.reshape(b * t, DIM)
    maskcol = input_mask.reshape(b * t, 1).astype(jnp.float32)
    maskf = jnp.broadcast_to(maskcol, (b * t, 128))
    cnt = jnp.maximum(jnp.sum(maskcol), 1.0).reshape(1, 1)
    keys_t = jnp.transpose(keys, (0, 2, 1, 3))
    ohi = jnp.asarray(_OHI)
    ohj = jnp.asarray(_OHJ)

    q, stats = _run_k1(xf, w_q, maskf)
    half = (b * t) // 2
    outs = []
    for s in (0, half):
        attn, vi = _run_k2(q[s:s + half], stats, bn_gamma.reshape(1, DIM),
                           bn_beta.reshape(1, DIM), maskf[s:s + half],
                           keys_t, ohi, ohj, cnt)
        outs.append(_run_k3(values, vi, attn))
    out = jnp.concatenate(outs, axis=0)
    return out.reshape(b, t, DIM)


# speedup vs baseline: 2.7140x; 1.1338x over previous
import functools

import jax
import jax.numpy as jnp
import numpy as np
from jax import lax
from jax.experimental import pallas as pl
from jax.experimental.pallas import tpu as pltpu
from jax.experimental.pallas import tpu_sc as plsc

DIM = 1024
HEADS = 4
NUM_KEYS = 256
TOPK = 32
SUB_D = 128
BN_EPS = 1e-5
N_CAND = 128

_TOKENS = 4096
_TILE = 256
_N_TILES = _TOKENS // _TILE


def _candidate_onehots():
    order = sorted(((i + 1) * (j + 1), i, j) for i in range(TOPK) for j in range(TOPK))
    pi = np.array([i for _, i, _ in order[:N_CAND]], dtype=np.int64)
    pj = np.array([j for _, _, j in order[:N_CAND]], dtype=np.int64)
    ohi = (np.arange(TOPK)[:, None] == pi[None, :]).astype(np.float32)
    ohj = (np.arange(TOPK)[:, None] == pj[None, :]).astype(np.float32)
    return ohi, ohj


_OHI, _OHJ = _candidate_onehots()


def _k1_body(x_ref, wq_ref, mask_ref, q_ref, stats_ref):
    i = pl.program_id(0)
    xt = x_ref[...].astype(jnp.bfloat16)
    q = lax.dot_general(xt, wq_ref[...].astype(jnp.bfloat16),
                        (((1,), (1,)), ((), ())),
                        preferred_element_type=jnp.float32)
    q_ref[...] = q
    m = mask_ref[:, 0:1]
    qm = q * m

    @pl.when(i == 0)
    def _():
        stats_ref[...] = jnp.zeros((8, DIM), jnp.float32)

    stats_ref[0:1, :] = stats_ref[0:1, :] + jnp.sum(qm, axis=0, keepdims=True)
    stats_ref[1:2, :] = stats_ref[1:2, :] + jnp.sum(qm * q, axis=0, keepdims=True)


def _run_k1(xf, w_q, maskf):
    return pl.pallas_call(
        _k1_body,
        grid=(_N_TILES,),
        in_specs=[
            pl.BlockSpec((_TILE, DIM), lambda i: (i, 0)),
            pl.BlockSpec((DIM, DIM), lambda i: (0, 0)),
            pl.BlockSpec((_TILE, 128), lambda i: (i, 0)),
        ],
        out_specs=[
            pl.BlockSpec((_TILE, DIM), lambda i: (i, 0)),
            pl.BlockSpec((8, DIM), lambda i: (0, 0)),
        ],
        out_shape=[
            jax.ShapeDtypeStruct((_TOKENS, DIM), jnp.float32),
            jax.ShapeDtypeStruct((8, DIM), jnp.float32),
        ],
    )(xf, w_q, maskf)


def _top32(work_ref, n_lanes):
    rows = work_ref.shape[0]
    lane = lax.broadcasted_iota(jnp.int32, (rows, n_lanes), 1)
    kcol = lax.broadcasted_iota(jnp.int32, (rows, TOPK), 1)

    def body(k, car):
        s_acc, i_acc = car
        w = work_ref[...]
        mx = jnp.max(w, axis=1, keepdims=True)
        am = jnp.min(jnp.where(w == mx, lane, n_lanes), axis=1, keepdims=True)
        work_ref[...] = jnp.where(lane == am, -3.0e38, w)
        s_acc = jnp.where(kcol == k, mx, s_acc)
        i_acc = jnp.where(kcol == k, am.astype(jnp.float32), i_acc)
        return s_acc, i_acc

    z = jnp.zeros((rows, TOPK), jnp.float32)
    return lax.fori_loop(0, TOPK, body, (z, z), unroll=4)


def _top32_with_payload(work_ref, payload):
    rows = work_ref.shape[0]
    n_lanes = N_CAND
    lane = lax.broadcasted_iota(jnp.int32, (rows, n_lanes), 1)
    kcol = lax.broadcasted_iota(jnp.int32, (rows, TOPK), 1)

    def body(k, car):
        s_acc, v_acc = car
        w = work_ref[...]
        mx = jnp.max(w, axis=1, keepdims=True)
        am = jnp.min(jnp.where(w == mx, lane, n_lanes), axis=1, keepdims=True)
        sel = lane == am
        work_ref[...] = jnp.where(sel, -3.0e38, w)
        vk = jnp.sum(jnp.where(sel, payload, 0.0), axis=1, keepdims=True)
        s_acc = jnp.where(kcol == k, mx, s_acc)
        v_acc = jnp.where(kcol == k, vk, v_acc)
        return s_acc, v_acc

    z = jnp.zeros((rows, TOPK), jnp.float32)
    return lax.fori_loop(0, TOPK, body, (z, z), unroll=4)


def _mm(a, b):
    return lax.dot_general(a, b, (((1,), (0,)), ((), ())),
                           preferred_element_type=jnp.float32,
                           precision=lax.Precision.HIGHEST)


def _k2_body(q_ref, stats_ref, gam_ref, bet_ref, mask_ref, keys_ref,
             ohi_ref, ohj_ref, cnt_ref, attn_ref, vi_ref, work_ref):
    cnt = cnt_ref[0, 0]
    s0 = stats_ref[0:1, :]
    s1 = stats_ref[1:2, :]
    mean = s0 / cnt
    var = s1 / cnt - mean * mean
    q = q_ref[...]
    m = mask_ref[:, 0:1]
    normed = (q - mean) / jnp.sqrt(var + BN_EPS) * gam_ref[...] + bet_ref[...]
    qn = jnp.where(m > 0, normed, q)
    ohi = ohi_ref[...]
    ohj = ohj_ref[...]

    for h in range(HEADS):
        tops = []
        for p in range(2):
            qhp = qn[:, p * 512 + h * 128:p * 512 + (h + 1) * 128]
            khp = keys_ref[h, p]
            work_ref[...] = lax.dot_general(
                qhp.astype(jnp.bfloat16), khp.astype(jnp.bfloat16),
                (((1,), (1,)), ((), ())),
                preferred_element_type=jnp.float32)
            tops.append(_top32(work_ref, NUM_KEYS))
        (sx, ix), (sy, iy) = tops
        csum = _mm(sx, ohi) + _mm(sy, ohj)
        cidx = _mm(ix * float(NUM_KEYS), ohi) + _mm(iy, ohj)
        work_ref[:, 0:N_CAND] = csum
        vs, vv = _top32_with_payload(work_ref.at[:, 0:N_CAND], cidx)
        mx0 = jnp.max(vs, axis=1, keepdims=True)
        e = jnp.exp(vs - mx0)
        a = e / jnp.sum(e, axis=1, keepdims=True)
        attn_ref[:, h * TOPK:(h + 1) * TOPK] = a
        vi_ref[:, h * TOPK:(h + 1) * TOPK] = vv.astype(jnp.int32)


def _run_k2(q, stats, gamma, beta, maskf, keys_t, ohi, ohj, cnt):
    n = q.shape[0]
    nt = n // _TILE
    return pl.pallas_call(
        _k2_body,
        grid=(nt,),
        in_specs=[
            pl.BlockSpec((_TILE, DIM), lambda i: (i, 0)),
            pl.BlockSpec((8, DIM), lambda i: (0, 0)),
            pl.BlockSpec((1, DIM), lambda i: (0, 0)),
            pl.BlockSpec((1, DIM), lambda i: (0, 0)),
            pl.BlockSpec((_TILE, 128), lambda i: (i, 0)),
            pl.BlockSpec((HEADS, 2, NUM_KEYS, SUB_D), lambda i: (0, 0, 0, 0)),
            pl.BlockSpec((TOPK, N_CAND), lambda i: (0, 0)),
            pl.BlockSpec((TOPK, N_CAND), lambda i: (0, 0)),
            pl.BlockSpec(memory_space=pltpu.SMEM),
        ],
        out_specs=[
            pl.BlockSpec((_TILE, HEADS * TOPK), lambda i: (i, 0)),
            pl.BlockSpec((_TILE, HEADS * TOPK), lambda i: (i, 0)),
        ],
        out_shape=[
            jax.ShapeDtypeStruct((n, HEADS * TOPK), jnp.float32),
            jax.ShapeDtypeStruct((n, HEADS * TOPK), jnp.int32),
        ],
        scratch_shapes=[pltpu.VMEM((_TILE, NUM_KEYS), jnp.float32)],
    )(q, stats, gamma, beta, maskf, keys_t, ohi, ohj, cnt)


_NC = 2
_NS = 16
_NW = _NC * _NS
_TPW = _TOKENS // _NW
_CH = 16
_NCH = (HEADS * TOPK) // _CH
_GT = _TPW * _NCH
_LSEG = DIM // 16


_TBLK = 32
_NBLK = _TPW // _TBLK
_NBUF = 2
_GBLK = _TBLK * _NCH


def _make_k3_body(tpw, nblk):
  def _k3_body(vals_hbm, vi_hbm, at_hbm, out_hbm,
               idx_v, w_v, rows_v, acc_v, sem0, sem1, sem2, sem3):
    wid = lax.axis_index("s") * _NC + lax.axis_index("c")
    base = wid * tpw
    sems = (sem0, sem1, sem2, sem3)

    def start_gather(tok, kc, b):
        pltpu.async_copy(vals_hbm.at[idx_v.at[tok, pl.ds(kc * _CH, _CH)]],
                         rows_v.at[b], sems[b])

    for blk in range(nblk):
        tok0 = base + blk * _TBLK
        pltpu.sync_copy(vi_hbm.at[pl.ds(tok0, _TBLK)], idx_v)
        pltpu.sync_copy(at_hbm.at[pl.ds(tok0, _TBLK)], w_v)
        for b in range(_NBUF):
            start_gather(b // _NCH, b % _NCH, b)

        @pl.loop(0, _GBLK, step=_NBUF)
        def _(g0):
            for b in range(_NBUF):
                g = g0 + b
                tok = g // _NCH
                kc = g % _NCH
                tm = lax.rem(tok, 8)
                pltpu.make_async_copy(
                    vals_hbm.at[idx_v.at[tok, pl.ds(kc * _CH, _CH)]],
                    rows_v.at[b], sems[b]).wait()

                @pl.when(kc == 0)
                def _():
                    @pl.loop(0, _LSEG, unroll=8)
                    def _(l):
                        acc_v[tm, pl.ds(l * 16, 16)] = jnp.zeros((16,),
                                                                 jnp.float32)

                wv = w_v[tok, pl.ds(kc * _CH, 16)]
                wlist = [wv[rr] for rr in range(_CH)]

                @pl.loop(0, _LSEG, unroll=4)
                def _(l):
                    seg = pl.ds(l * 16, 16)
                    v = wlist[0] * rows_v[b, 0, seg]
                    for rr in range(1, _CH):
                        v = v + wlist[rr] * rows_v[b, rr, seg]
                    plsc.addupdate(acc_v.at[tm, seg], v)

                @pl.when((kc == _NCH - 1) & (tm == 7))
                def _():
                    off = pl.multiple_of(tok0 + tok - 7, 8)
                    pltpu.sync_copy(acc_v, out_hbm.at[pl.ds(off, 8)])

                @pl.when(g + _NBUF < _GBLK)
                def _():
                    g2 = g + _NBUF
                    start_gather(g2 // _NCH, lax.rem(g2, _NCH), b)

  return _k3_body


def _run_k3(values, vi3, attn):
    n = vi3.shape[0]
    tpw = n // _NW
    nblk = tpw // _TBLK
    mesh = plsc.VectorSubcoreMesh(core_axis_name="c", subcore_axis_name="s")
    k = functools.partial(
        pl.kernel,
        out_type=jax.ShapeDtypeStruct((n, DIM), jnp.float32),
        mesh=mesh,
        scratch_types=[
            pltpu.VMEM((_TBLK, HEADS * TOPK), jnp.int32),
            pltpu.VMEM((_TBLK, HEADS * TOPK), jnp.float32),
            pltpu.VMEM((_NBUF, _CH, DIM), jnp.float32),
            pltpu.VMEM((8, DIM), jnp.float32),
            pltpu.SemaphoreType.DMA,
            pltpu.SemaphoreType.DMA,
            pltpu.SemaphoreType.DMA,
            pltpu.SemaphoreType.DMA,
        ],
    )(_make_k3_body(tpw, nblk))
    return k(values, vi3, attn)


def kernel(x, input_mask, w_q, bn_gamma, bn_beta, keys, values):
    b, t = x.shape[0], x.shape[1]
    xf = x.reshape(b * t, DIM)
    maskcol = input_mask.reshape(b * t, 1).astype(jnp.float32)
    maskf = jnp.broadcast_to(maskcol, (b * t, 128))
    cnt = jnp.maximum(jnp.sum(maskcol), 1.0).reshape(1, 1)
    keys_t = jnp.transpose(keys, (0, 2, 1, 3))
    ohi = jnp.asarray(_OHI)
    ohj = jnp.asarray(_OHJ)

    q, stats = _run_k1(xf, w_q, maskf)
    half = (b * t) // 2
    outs = []
    for s in (0, half):
        attn, vi = _run_k2(q[s:s + half], stats, bn_gamma.reshape(1, DIM),
                           bn_beta.reshape(1, DIM), maskf[s:s + half],
                           keys_t, ohi, ohj, cnt)
        outs.append(_run_k3(values, vi, attn))
    out = jnp.concatenate(outs, axis=0)
    return out.reshape(b, t, DIM)


# speedup vs baseline: 3.2273x; 1.1891x over previous
import functools

import jax
import jax.numpy as jnp
import numpy as np
from jax import lax
from jax.experimental import pallas as pl
from jax.experimental.pallas import tpu as pltpu
from jax.experimental.pallas import tpu_sc as plsc

DIM = 1024
HEADS = 4
NUM_KEYS = 256
TOPK = 32
SUB_D = 128
BN_EPS = 1e-5
N_CAND = 128

_TOKENS = 4096
_TILE = 256
_N_TILES = _TOKENS // _TILE


def _candidate_onehots():
    order = sorted(((i + 1) * (j + 1), i, j) for i in range(TOPK) for j in range(TOPK))
    pi = np.array([i for _, i, _ in order[:N_CAND]], dtype=np.int64)
    pj = np.array([j for _, _, j in order[:N_CAND]], dtype=np.int64)
    ohi = (np.arange(TOPK)[:, None] == pi[None, :]).astype(np.float32)
    ohj = (np.arange(TOPK)[:, None] == pj[None, :]).astype(np.float32)
    return ohi, ohj


_OHI, _OHJ = _candidate_onehots()


def _k1_body(x_ref, wq_ref, mask_ref, q_ref, stats_ref):
    i = pl.program_id(0)
    xt = x_ref[...].astype(jnp.bfloat16)
    q = lax.dot_general(xt, wq_ref[...].astype(jnp.bfloat16),
                        (((1,), (1,)), ((), ())),
                        preferred_element_type=jnp.float32)
    q_ref[...] = q
    m = mask_ref[:, 0:1]
    qm = q * m

    @pl.when(i == 0)
    def _():
        stats_ref[...] = jnp.zeros((8, DIM), jnp.float32)

    stats_ref[0:1, :] = stats_ref[0:1, :] + jnp.sum(qm, axis=0, keepdims=True)
    stats_ref[1:2, :] = stats_ref[1:2, :] + jnp.sum(qm * q, axis=0, keepdims=True)


def _run_k1(xf, w_q, maskf):
    return pl.pallas_call(
        _k1_body,
        grid=(_N_TILES,),
        in_specs=[
            pl.BlockSpec((_TILE, DIM), lambda i: (i, 0)),
            pl.BlockSpec((DIM, DIM), lambda i: (0, 0)),
            pl.BlockSpec((_TILE, 128), lambda i: (i, 0)),
        ],
        out_specs=[
            pl.BlockSpec((_TILE, DIM), lambda i: (i, 0)),
            pl.BlockSpec((8, DIM), lambda i: (0, 0)),
        ],
        out_shape=[
            jax.ShapeDtypeStruct((_TOKENS, DIM), jnp.float32),
            jax.ShapeDtypeStruct((8, DIM), jnp.float32),
        ],
    )(xf, w_q, maskf)


def _top32(work_ref, n_lanes):
    rows = work_ref.shape[0]
    lane = lax.broadcasted_iota(jnp.int32, (rows, n_lanes), 1)
    kcol = lax.broadcasted_iota(jnp.int32, (rows, TOPK), 1)

    def body(k, car):
        s_acc, i_acc = car
        w = work_ref[...]
        mx = jnp.max(w, axis=1, keepdims=True)
        am = jnp.min(jnp.where(w == mx, lane, n_lanes), axis=1, keepdims=True)
        work_ref[...] = jnp.where(lane == am, -3.0e38, w)
        s_acc = jnp.where(kcol == k, mx, s_acc)
        i_acc = jnp.where(kcol == k, am.astype(jnp.float32), i_acc)
        return s_acc, i_acc

    z = jnp.zeros((rows, TOPK), jnp.float32)
    return lax.fori_loop(0, TOPK, body, (z, z), unroll=4)


def _top32_with_payload(work_ref, payload):
    rows = work_ref.shape[0]
    n_lanes = N_CAND
    lane = lax.broadcasted_iota(jnp.int32, (rows, n_lanes), 1)
    kcol = lax.broadcasted_iota(jnp.int32, (rows, TOPK), 1)

    def body(k, car):
        s_acc, v_acc = car
        w = work_ref[...]
        mx = jnp.max(w, axis=1, keepdims=True)
        am = jnp.min(jnp.where(w == mx, lane, n_lanes), axis=1, keepdims=True)
        sel = lane == am
        work_ref[...] = jnp.where(sel, -3.0e38, w)
        vk = jnp.sum(jnp.where(sel, payload, 0.0), axis=1, keepdims=True)
        s_acc = jnp.where(kcol == k, mx, s_acc)
        v_acc = jnp.where(kcol == k, vk, v_acc)
        return s_acc, v_acc

    z = jnp.zeros((rows, TOPK), jnp.float32)
    return lax.fori_loop(0, TOPK, body, (z, z), unroll=4)


def _mm(a, b):
    return lax.dot_general(a, b, (((1,), (0,)), ((), ())),
                           preferred_element_type=jnp.float32,
                           precision=lax.Precision.HIGHEST)


def _k2_body(q_ref, stats_ref, gam_ref, bet_ref, mask_ref, keys_ref,
             ohi_ref, ohj_ref, cnt_ref, attn_ref, vi_ref, work_ref):
    cnt = cnt_ref[0, 0]
    s0 = stats_ref[0:1, :]
    s1 = stats_ref[1:2, :]
    mean = s0 / cnt
    var = s1 / cnt - mean * mean
    q = q_ref[...]
    m = mask_ref[:, 0:1]
    normed = (q - mean) / jnp.sqrt(var + BN_EPS) * gam_ref[...] + bet_ref[...]
    qn = jnp.where(m > 0, normed, q)
    ohi = ohi_ref[...]
    ohj = ohj_ref[...]

    for h in range(HEADS):
        tops = []
        for p in range(2):
            qhp = qn[:, p * 512 + h * 128:p * 512 + (h + 1) * 128]
            khp = keys_ref[h, p]
            work_ref[...] = lax.dot_general(
                qhp.astype(jnp.bfloat16), khp.astype(jnp.bfloat16),
                (((1,), (1,)), ((), ())),
                preferred_element_type=jnp.float32)
            tops.append(_top32(work_ref, NUM_KEYS))
        (sx, ix), (sy, iy) = tops
        csum = _mm(sx, ohi) + _mm(sy, ohj)
        cidx = _mm(ix * float(NUM_KEYS), ohi) + _mm(iy, ohj)
        work_ref[:, 0:N_CAND] = csum
        vs, vv = _top32_with_payload(work_ref.at[:, 0:N_CAND], cidx)
        mx0 = jnp.max(vs, axis=1, keepdims=True)
        e = jnp.exp(vs - mx0)
        a = e / jnp.sum(e, axis=1, keepdims=True)
        attn_ref[:, h * TOPK:(h + 1) * TOPK] = a
        vi_ref[:, h * TOPK:(h + 1) * TOPK] = vv.astype(jnp.int32)


def _run_k2(q, stats, gamma, beta, maskf, keys_t, ohi, ohj, cnt):
    n = q.shape[0]
    nt = n // _TILE
    return pl.pallas_call(
        _k2_body,
        grid=(nt,),
        in_specs=[
            pl.BlockSpec((_TILE, DIM), lambda i: (i, 0)),
            pl.BlockSpec((8, DIM), lambda i: (0, 0)),
            pl.BlockSpec((1, DIM), lambda i: (0, 0)),
            pl.BlockSpec((1, DIM), lambda i: (0, 0)),
            pl.BlockSpec((_TILE, 128), lambda i: (i, 0)),
            pl.BlockSpec((HEADS, 2, NUM_KEYS, SUB_D), lambda i: (0, 0, 0, 0)),
            pl.BlockSpec((TOPK, N_CAND), lambda i: (0, 0)),
            pl.BlockSpec((TOPK, N_CAND), lambda i: (0, 0)),
            pl.BlockSpec(memory_space=pltpu.SMEM),
        ],
        out_specs=[
            pl.BlockSpec((_TILE, HEADS * TOPK), lambda i: (i, 0)),
            pl.BlockSpec((_TILE, HEADS * TOPK), lambda i: (i, 0)),
        ],
        out_shape=[
            jax.ShapeDtypeStruct((n, HEADS * TOPK), jnp.float32),
            jax.ShapeDtypeStruct((n, HEADS * TOPK), jnp.int32),
        ],
        scratch_shapes=[pltpu.VMEM((_TILE, NUM_KEYS), jnp.float32)],
    )(q, stats, gamma, beta, maskf, keys_t, ohi, ohj, cnt)


_NC = 2
_NS = 16
_NW = _NC * _NS
_TPW = _TOKENS // _NW
_CH = 16
_NCH = (HEADS * TOPK) // _CH
_GT = _TPW * _NCH
_LSEG = DIM // 16


_TBLK = 32
_NBLK = _TPW // _TBLK
_NBUF = 2
_GBLK = _TBLK * _NCH


def _make_k3_body(tpw, nblk):
  def _k3_body(vals_hbm, vi_hbm, at_hbm, out_hbm,
               idx_v, w_v, rows_v, acc_v, sem0, sem1, sem2, sem3):
    wid = lax.axis_index("s") * _NC + lax.axis_index("c")
    base = wid * tpw
    sems = (sem0, sem1, sem2, sem3)

    def start_gather(tok, kc, b):
        pltpu.async_copy(vals_hbm.at[idx_v.at[tok, pl.ds(kc * _CH, _CH)]],
                         rows_v.at[b], sems[b])

    for blk in range(nblk):
        tok0 = base + blk * _TBLK
        pltpu.sync_copy(vi_hbm.at[pl.ds(tok0, _TBLK)], idx_v)
        pltpu.sync_copy(at_hbm.at[pl.ds(tok0, _TBLK)], w_v)
        for b in range(_NBUF):
            start_gather(b // _NCH, b % _NCH, b)

        @pl.loop(0, _GBLK, step=_NBUF)
        def _(g0):
            for b in range(_NBUF):
                g = g0 + b
                tok = g // _NCH
                kc = g % _NCH
                tm = lax.rem(tok, 8)
                pltpu.make_async_copy(
                    vals_hbm.at[idx_v.at[tok, pl.ds(kc * _CH, _CH)]],
                    rows_v.at[b], sems[b]).wait()

                @pl.when(kc == 0)
                def _():
                    @pl.loop(0, _LSEG, unroll=8)
                    def _(l):
                        acc_v[tm, pl.ds(l * 16, 16)] = jnp.zeros((16,),
                                                                 jnp.float32)

                wv = w_v[tok, pl.ds(kc * _CH, 16)]
                wlist = [wv[rr] for rr in range(_CH)]

                @pl.loop(0, _LSEG, unroll=4)
                def _(l):
                    seg = pl.ds(l * 16, 16)
                    v = wlist[0] * rows_v[b, 0, seg]
                    for rr in range(1, _CH):
                        v = v + wlist[rr] * rows_v[b, rr, seg]
                    plsc.addupdate(acc_v.at[tm, seg], v)

                @pl.when((kc == _NCH - 1) & (tm == 7))
                def _():
                    off = pl.multiple_of(tok0 + tok - 7, 8)
                    pltpu.sync_copy(acc_v, out_hbm.at[pl.ds(off, 8)])

                @pl.when(g + _NBUF < _GBLK)
                def _():
                    g2 = g + _NBUF
                    start_gather(g2 // _NCH, lax.rem(g2, _NCH), b)

  return _k3_body


def _run_k3(values, vi3, attn):
    n = vi3.shape[0]
    tpw = n // _NW
    nblk = tpw // _TBLK
    mesh = plsc.VectorSubcoreMesh(core_axis_name="c", subcore_axis_name="s")
    k = functools.partial(
        pl.kernel,
        out_type=jax.ShapeDtypeStruct((n, DIM), jnp.float32),
        mesh=mesh,
        scratch_types=[
            pltpu.VMEM((_TBLK, HEADS * TOPK), jnp.int32),
            pltpu.VMEM((_TBLK, HEADS * TOPK), jnp.float32),
            pltpu.VMEM((_NBUF, _CH, DIM), jnp.float32),
            pltpu.VMEM((8, DIM), jnp.float32),
            pltpu.SemaphoreType.DMA,
            pltpu.SemaphoreType.DMA,
            pltpu.SemaphoreType.DMA,
            pltpu.SemaphoreType.DMA,
        ],
    )(_make_k3_body(tpw, nblk))
    return k(values, vi3, attn)


def kernel(x, input_mask, w_q, bn_gamma, bn_beta, keys, values):
    b, t = x.shape[0], x.shape[1]
    xf = x.reshape(b * t, DIM)
    maskcol = input_mask.reshape(b * t, 1).astype(jnp.float32)
    maskf = jnp.broadcast_to(maskcol, (b * t, 128))
    cnt = jnp.maximum(jnp.sum(maskcol), 1.0).reshape(1, 1)
    keys_t = jnp.transpose(keys, (0, 2, 1, 3))
    ohi = jnp.asarray(_OHI)
    ohj = jnp.asarray(_OHJ)

    q, stats = _run_k1(xf, w_q, maskf)
    part = (b * t) // 4
    outs = []
    for s in range(0, b * t, part):
        attn, vi = _run_k2(q[s:s + part], stats, bn_gamma.reshape(1, DIM),
                           bn_beta.reshape(1, DIM), maskf[s:s + part],
                           keys_t, ohi, ohj, cnt)
        outs.append(_run_k3(values, vi, attn))
    out = jnp.concatenate(outs, axis=0)
    return out.reshape(b, t, DIM)
